# baseline (speedup 1.0000x reference)
import jax
import jax.numpy as jnp
from jax import lax
from jax.experimental import pallas as pl
from jax.experimental.pallas import tpu as pltpu
from jax.experimental.pallas import tpu_sc as plsc

ED = 64
CTX = 20
K = 20
NC = 2
NS = 16
NWK = NC * NS
CHUNK = 16
LANES = 16
PIECES = ((0, 128), (128, 128), (256, 64))

_params = pltpu.CompilerParams(
    needs_layout_passes=False, use_tc_tiling_on_sc=False)


TP3 = 2608
NP3 = 5
W3 = TP3 * NP3
NCG = 8


def _tr_body(src_hbm, dst_hbm, slab_a, slab_b, otile, sem_a, sem_b, sem_o):
    wid = lax.axis_index("s") * NC + lax.axis_index("c")
    lanes = jnp.arange(LANES, dtype=jnp.int32)
    nw = src_hbm.shape[1]
    d0 = (wid % 4) * LANES
    cg = wid // 4
    colshare = -(-nw // NCG)
    c0 = jnp.minimum((cg * colshare) // 8 * 8, (nw - W3) // 8 * 8)

    def start(p, slab, sem):
        off = pl.multiple_of(c0 + p * TP3, 8)
        pltpu.async_copy(src_hbm.at[pl.ds(d0, LANES), pl.ds(off, TP3)],
                         slab, sem)

    def wait_in(slab, sem):
        off = pl.multiple_of(c0, 8)
        pltpu.make_async_copy(src_hbm.at[pl.ds(d0, LANES), pl.ds(off, TP3)],
                              slab, sem).wait()

    def out_slice(p):
        return dst_hbm.at[pl.ds(c0 + p * TP3, TP3), pl.ds(d0, LANES)]

    def wait_out():
        pltpu.make_async_copy(otile, out_slice(0), sem_o).wait()

    def transpose_piece(slab):
        def w_body(wb, carry):
            w0 = wb * LANES
            for u in range(LANES):
                vals = plsc.load_gather(
                    slab, [lanes, jnp.full((LANES,), w0 + u, jnp.int32)])
                otile[w0 + u, pl.ds(0, LANES)] = vals
            return carry
        lax.fori_loop(0, TP3 // LANES, w_body, 0)

    bufs = ((slab_a, sem_a), (slab_b, sem_b))
    start(0, slab_a, sem_a)
    for p in range(NP3):
        slab, sem = bufs[p % 2]
        wait_in(slab, sem)
        if p + 1 < NP3:
            start(p + 1, *bufs[(p + 1) % 2])
        if p >= 1:
            wait_out()
        transpose_piece(slab)
        pltpu.async_copy(otile, out_slice(p), sem_o)
    wait_out()


def _make_transpose(nw):
    mesh = plsc.VectorSubcoreMesh(core_axis_name="c", subcore_axis_name="s")
    return pl.kernel(
        _tr_body,
        out_type=jax.ShapeDtypeStruct((nw, ED), jnp.float32),
        mesh=mesh,
        scratch_types=[
            pltpu.VMEM((LANES, TP3), jnp.float32),
            pltpu.VMEM((LANES, TP3), jnp.float32),
            pltpu.VMEM((TP3, LANES), jnp.float32),
            pltpu.SemaphoreType.DMA,
            pltpu.SemaphoreType.DMA,
            pltpu.SemaphoreType.DMA,
        ],
        compiler_params=_params,
    )


def _dm_body(ctx_hbm, doc_hbm, tn_hbm, d_hbm, w_hbm, wpt_hbm, out_hbm,
             ctx_idx, tn_idx, doc_idx,
             gc_a, gt_a, gd_a, gc_b, gt_b, gd_b, out_v, sem_a, sem_b):
    wid = lax.axis_index("s") * NC + lax.axis_index("c")
    nb = doc_hbm.shape[0] // NWK
    b0 = wid * nb

    h1 = pltpu.async_copy(ctx_hbm.at[pl.ds(b0 * CTX, nb * CTX)], ctx_idx,
                          sem_a)
    h2 = pltpu.async_copy(tn_hbm.at[pl.ds(b0 * K, nb * K)], tn_idx, sem_a)
    h3 = pltpu.async_copy(doc_hbm.at[pl.ds(b0, nb)], doc_idx, sem_a)
    h1.wait()
    h2.wait()
    h3.wait()

    lanes = jnp.arange(LANES, dtype=jnp.int32)
    xor_perms = [lanes ^ sh for sh in (8, 4, 2, 1)]
    nchunks = nb // CHUNK

    def lane_sum(v):
        for perm in xor_perms:
            v = v + v.at[perm].get(mode="promise_in_bounds")
        return v

    def issue(c, gc, gt, gd, sem):
        ib = c * (CHUNK * CTX)
        for off, n in PIECES:
            pltpu.async_copy(w_hbm.at[ctx_idx.at[pl.ds(ib + off, n)]],
                             gc.at[pl.ds(off, n)], sem)
            pltpu.async_copy(wpt_hbm.at[tn_idx.at[pl.ds(ib + off, n)]],
                             gt.at[pl.ds(off, n)], sem)
        pltpu.async_copy(d_hbm.at[doc_idx.at[pl.ds(c * CHUNK, CHUNK)]],
                         gd, sem)

    def drain(gc, gt, gd, sem):
        for off, n in PIECES:
            pltpu.make_async_copy(w_hbm.at[ctx_idx.at[pl.ds(0, n)]],
                                  gc.at[pl.ds(off, n)], sem).wait()
            pltpu.make_async_copy(wpt_hbm.at[tn_idx.at[pl.ds(0, n)]],
                                  gt.at[pl.ds(off, n)], sem).wait()
        pltpu.make_async_copy(d_hbm.at[doc_idx.at[pl.ds(0, CHUNK)]],
                              gd, sem).wait()

    def compute(c, gc, gt, gd):
        o0 = c * (CHUNK * K)

        def b_body(i, carry2):
            r0 = i * CTX
            acc = [gd[i, pl.ds(v * LANES, LANES)] for v in range(4)]
            for j in range(CTX):
                for v in range(4):
                    acc[v] = acc[v] + gc[r0 + j, pl.ds(v * LANES, LANES)]
            ov0 = jnp.zeros((LANES,), jnp.float32)
            ov1 = jnp.zeros((LANES,), jnp.float32)
            for k in range(K):
                p = acc[0] * gt[r0 + k, pl.ds(0, LANES)]
                for v in range(1, 4):
                    p = p + acc[v] * gt[r0 + k, pl.ds(v * LANES, LANES)]
                sv = lane_sum(p)
                if k < LANES:
                    ov0 = jnp.where(lanes == k, sv, ov0)
                else:
                    ov1 = jnp.where(lanes == (k - LANES), sv, ov1)
            out_v[pl.ds(o0 + i * K, LANES)] = ov0
            out_v[pl.ds(o0 + i * K + LANES, LANES)] = ov1
            return carry2

        lax.fori_loop(0, CHUNK, b_body, 0)

    issue(0, gc_a, gt_a, gd_a, sem_a)
    nsteps = nchunks // 2

    def step(t, carry):
        c = 2 * t
        drain(gc_a, gt_a, gd_a, sem_a)
        issue(c + 1, gc_b, gt_b, gd_b, sem_b)
        compute(c, gc_a, gt_a, gd_a)
        drain(gc_b, gt_b, gd_b, sem_b)

        @pl.when(t < nsteps - 1)
        def _prefetch():
            issue(c + 2, gc_a, gt_a, gd_a, sem_a)

        compute(c + 1, gc_b, gt_b, gd_b)
        return carry

    lax.fori_loop(0, nsteps, step, 0)
    pltpu.sync_copy(out_v.at[pl.ds(0, nb * K)],
                    out_hbm.at[pl.ds(b0 * K, nb * K)])


def _make_kernel(B):
    nb = B // NWK
    mesh = plsc.VectorSubcoreMesh(core_axis_name="c", subcore_axis_name="s")
    gather_bufs = [
        pltpu.VMEM((CHUNK * CTX, ED), jnp.float32),
        pltpu.VMEM((CHUNK * K, ED), jnp.float32),
        pltpu.VMEM((CHUNK, ED), jnp.float32),
    ]
    return pl.kernel(
        _dm_body,
        out_type=jax.ShapeDtypeStruct((B * K,), jnp.float32),
        mesh=mesh,
        scratch_types=[
            pltpu.VMEM((nb * CTX,), jnp.int32),
            pltpu.VMEM((nb * K,), jnp.int32),
            pltpu.VMEM((nb,), jnp.int32),
            *gather_bufs,
            *gather_bufs,
            pltpu.VMEM((nb * K + LANES,), jnp.float32),
            pltpu.SemaphoreType.DMA,
            pltpu.SemaphoreType.DMA,
        ],
        compiler_params=pltpu.CompilerParams(
            needs_layout_passes=False, use_tc_tiling_on_sc=False),
    )


def kernel(ctx_ids, doc_ids, target_and_noise_ids, D, W, Wp):
    B = ctx_ids.shape[0]
    WpT = _make_transpose(Wp.shape[1])(Wp)
    out = _make_kernel(B)(
        ctx_ids.reshape(-1), doc_ids, target_and_noise_ids.reshape(-1),
        D, W, WpT)
    return out.reshape(B, K)


# speedup vs baseline: 1.0158x; 1.0158x over previous
import jax
import jax.numpy as jnp
from jax import lax
from jax.experimental import pallas as pl
from jax.experimental.pallas import tpu as pltpu
from jax.experimental.pallas import tpu_sc as plsc

ED = 64
CTX = 20
K = 20
NC = 2
NS = 16
NWK = NC * NS
CHUNK = 16
LANES = 16
PIECES = ((0, 128), (128, 128), (256, 64))

_params = pltpu.CompilerParams(
    needs_layout_passes=False, use_tc_tiling_on_sc=False)


TP3 = 2608
NP3 = 5
W3 = TP3 * NP3
NCG = 8


def _tr_body(src_hbm, dst_hbm, slab_a, slab_b, otile, sem_a, sem_b, sem_o):
    wid = lax.axis_index("s") * NC + lax.axis_index("c")
    lanes = jnp.arange(LANES, dtype=jnp.int32)
    nw = src_hbm.shape[1]
    d0 = (wid % 4) * LANES
    cg = wid // 4
    colshare = -(-nw // NCG)
    c0 = jnp.minimum((cg * colshare) // 8 * 8, (nw - W3) // 8 * 8)

    def start(p, slab, sem):
        off = pl.multiple_of(c0 + p * TP3, 8)
        pltpu.async_copy(src_hbm.at[pl.ds(d0, LANES), pl.ds(off, TP3)],
                         slab.at[:, pl.ds(0, TP3)], sem)

    def wait_in(slab, sem):
        off = pl.multiple_of(c0, 8)
        pltpu.make_async_copy(src_hbm.at[pl.ds(d0, LANES), pl.ds(off, TP3)],
                              slab.at[:, pl.ds(0, TP3)], sem).wait()

    def out_slice(p):
        return dst_hbm.at[pl.ds(c0 + p * TP3, TP3), pl.ds(d0, LANES)]

    def wait_out():
        pltpu.make_async_copy(otile, out_slice(0), sem_o).wait()

    def transpose_piece(slab):
        def w_body(wb, carry):
            w0 = wb * LANES
            for u in range(LANES):
                vals = plsc.load_gather(
                    slab, [lanes, jnp.full((LANES,), w0 + u, jnp.int32)])
                otile[w0 + u, pl.ds(0, LANES)] = vals
            return carry
        lax.fori_loop(0, TP3 // LANES, w_body, 0)

    bufs = ((slab_a, sem_a), (slab_b, sem_b))
    start(0, slab_a, sem_a)
    for p in range(NP3):
        slab, sem = bufs[p % 2]
        wait_in(slab, sem)
        if p + 1 < NP3:
            start(p + 1, *bufs[(p + 1) % 2])
        if p >= 1:
            wait_out()
        transpose_piece(slab)
        pltpu.async_copy(otile, out_slice(p), sem_o)
    wait_out()


def _make_transpose(nw):
    mesh = plsc.VectorSubcoreMesh(core_axis_name="c", subcore_axis_name="s")
    return pl.kernel(
        _tr_body,
        out_type=jax.ShapeDtypeStruct((nw, ED), jnp.float32),
        mesh=mesh,
        scratch_types=[
            pltpu.VMEM((LANES, TP3 + 1), jnp.float32),
            pltpu.VMEM((LANES, TP3 + 1), jnp.float32),
            pltpu.VMEM((TP3, LANES), jnp.float32),
            pltpu.SemaphoreType.DMA,
            pltpu.SemaphoreType.DMA,
            pltpu.SemaphoreType.DMA,
        ],
        compiler_params=_params,
    )


def _dm_body(ctx_hbm, doc_hbm, tn_hbm, d_hbm, w_hbm, wpt_hbm, out_hbm,
             ctx_idx, tn_idx, doc_idx,
             gc_a, gt_a, gd_a, gc_b, gt_b, gd_b, out_v, sem_a, sem_b):
    wid = lax.axis_index("s") * NC + lax.axis_index("c")
    nb = doc_hbm.shape[0] // NWK
    b0 = wid * nb

    h1 = pltpu.async_copy(ctx_hbm.at[pl.ds(b0 * CTX, nb * CTX)], ctx_idx,
                          sem_a)
    h2 = pltpu.async_copy(tn_hbm.at[pl.ds(b0 * K, nb * K)], tn_idx, sem_a)
    h3 = pltpu.async_copy(doc_hbm.at[pl.ds(b0, nb)], doc_idx, sem_a)
    h1.wait()
    h2.wait()
    h3.wait()

    lanes = jnp.arange(LANES, dtype=jnp.int32)
    xor_perms = [lanes ^ sh for sh in (8, 4, 2, 1)]
    nchunks = nb // CHUNK

    def lane_sum(v):
        for perm in xor_perms:
            v = v + v.at[perm].get(mode="promise_in_bounds")
        return v

    def issue(c, gc, gt, gd, sem):
        ib = c * (CHUNK * CTX)
        for off, n in PIECES:
            pltpu.async_copy(w_hbm.at[ctx_idx.at[pl.ds(ib + off, n)]],
                             gc.at[pl.ds(off, n)], sem)
            pltpu.async_copy(wpt_hbm.at[tn_idx.at[pl.ds(ib + off, n)]],
                             gt.at[pl.ds(off, n)], sem)
        pltpu.async_copy(d_hbm.at[doc_idx.at[pl.ds(c * CHUNK, CHUNK)]],
                         gd, sem)

    def drain(gc, gt, gd, sem):
        for off, n in PIECES:
            pltpu.make_async_copy(w_hbm.at[ctx_idx.at[pl.ds(0, n)]],
                                  gc.at[pl.ds(off, n)], sem).wait()
            pltpu.make_async_copy(wpt_hbm.at[tn_idx.at[pl.ds(0, n)]],
                                  gt.at[pl.ds(off, n)], sem).wait()
        pltpu.make_async_copy(d_hbm.at[doc_idx.at[pl.ds(0, CHUNK)]],
                              gd, sem).wait()

    def compute(c, gc, gt, gd):
        o0 = c * (CHUNK * K)

        def b_body(i, carry2):
            r0 = i * CTX
            acc = [gd[i, pl.ds(v * LANES, LANES)] for v in range(4)]
            for j in range(CTX):
                for v in range(4):
                    acc[v] = acc[v] + gc[r0 + j, pl.ds(v * LANES, LANES)]
            ov0 = jnp.zeros((LANES,), jnp.float32)
            ov1 = jnp.zeros((LANES,), jnp.float32)
            for k in range(K):
                p = acc[0] * gt[r0 + k, pl.ds(0, LANES)]
                for v in range(1, 4):
                    p = p + acc[v] * gt[r0 + k, pl.ds(v * LANES, LANES)]
                sv = lane_sum(p)
                if k < LANES:
                    ov0 = jnp.where(lanes == k, sv, ov0)
                else:
                    ov1 = jnp.where(lanes == (k - LANES), sv, ov1)
            out_v[pl.ds(o0 + i * K, LANES)] = ov0
            out_v[pl.ds(o0 + i * K + LANES, LANES)] = ov1
            return carry2

        lax.fori_loop(0, CHUNK, b_body, 0)

    issue(0, gc_a, gt_a, gd_a, sem_a)
    nsteps = nchunks // 2

    def step(t, carry):
        c = 2 * t
        drain(gc_a, gt_a, gd_a, sem_a)
        issue(c + 1, gc_b, gt_b, gd_b, sem_b)
        compute(c, gc_a, gt_a, gd_a)
        drain(gc_b, gt_b, gd_b, sem_b)

        @pl.when(t < nsteps - 1)
        def _prefetch():
            issue(c + 2, gc_a, gt_a, gd_a, sem_a)

        compute(c + 1, gc_b, gt_b, gd_b)
        return carry

    lax.fori_loop(0, nsteps, step, 0)
    pltpu.sync_copy(out_v.at[pl.ds(0, nb * K)],
                    out_hbm.at[pl.ds(b0 * K, nb * K)])


def _make_kernel(B):
    nb = B // NWK
    mesh = plsc.VectorSubcoreMesh(core_axis_name="c", subcore_axis_name="s")
    gather_bufs = [
        pltpu.VMEM((CHUNK * CTX, ED), jnp.float32),
        pltpu.VMEM((CHUNK * K, ED), jnp.float32),
        pltpu.VMEM((CHUNK, ED), jnp.float32),
    ]
    return pl.kernel(
        _dm_body,
        out_type=jax.ShapeDtypeStruct((B * K,), jnp.float32),
        mesh=mesh,
        scratch_types=[
            pltpu.VMEM((nb * CTX,), jnp.int32),
            pltpu.VMEM((nb * K,), jnp.int32),
            pltpu.VMEM((nb,), jnp.int32),
            *gather_bufs,
            *gather_bufs,
            pltpu.VMEM((nb * K + LANES,), jnp.float32),
            pltpu.SemaphoreType.DMA,
            pltpu.SemaphoreType.DMA,
        ],
        compiler_params=pltpu.CompilerParams(
            needs_layout_passes=False, use_tc_tiling_on_sc=False),
    )


def kernel(ctx_ids, doc_ids, target_and_noise_ids, D, W, Wp):
    B = ctx_ids.shape[0]
    WpT = _make_transpose(Wp.shape[1])(Wp)
    out = _make_kernel(B)(
        ctx_ids.reshape(-1), doc_ids, target_and_noise_ids.reshape(-1),
        D, W, WpT)
    return out.reshape(B, K)
